# baseline (speedup 1.0000x reference)
import jax
import jax.numpy as jnp
import numpy as np
from jax.experimental import pallas as pl
from jax.experimental.pallas import tpu as pltpu
from jax.sharding import Mesh, PartitionSpec as P

K_BANK = 2048
BLOCK_N = 8192
CHUNK = 512


def _hist(scores):
    m = jnp.max(scores, axis=-1, keepdims=True)
    return jnp.sum((scores == m).astype(jnp.int32), axis=0, keepdims=True)


def _fused_body(x_ref, w_ref, bt_ref, o_ref, m_ref):
    i = pl.program_id(0)

    @pl.when(i == 0)
    def _merge():
        m_ref[:] = jnp.dot(w_ref[:], bt_ref[:], preferred_element_type=jnp.float32)

    nchunk = BLOCK_N // CHUNK

    def _mm(c):
        x = x_ref[c * CHUNK:(c + 1) * CHUNK, :]
        return jnp.dot(x, m_ref[:], preferred_element_type=jnp.float32)

    LOOKAHEAD = 2
    pending = [_mm(c) for c in range(min(LOOKAHEAD, nchunk))]
    part = jnp.zeros((1, K_BANK), jnp.int32)
    for c in range(nchunk):
        if c + LOOKAHEAD < nchunk:
            pending.append(_mm(c + LOOKAHEAD))
        part += _hist(pending.pop(0))

    @pl.when(i == 0)
    def _init():
        o_ref[:] = part

    @pl.when(i > 0)
    def _acc():
        o_ref[:] += part


def _count_rows(images, W, bank_t):
    n = images.shape[0]
    grid = (n // BLOCK_N,)
    counts = pl.pallas_call(
        _fused_body,
        grid=grid,
        in_specs=[
            pl.BlockSpec((BLOCK_N, images.shape[1]), lambda i: (i, 0)),
            pl.BlockSpec(W.shape, lambda i: (0, 0)),
            pl.BlockSpec(bank_t.shape, lambda i: (0, 0)),
        ],
        out_specs=pl.BlockSpec((1, K_BANK), lambda i: (0, 0)),
        out_shape=jax.ShapeDtypeStruct((1, K_BANK), jnp.int32),
        scratch_shapes=[pltpu.VMEM((64, K_BANK), jnp.float32)],
    )(images, W, bank_t)
    return counts


def kernel(images, W, bank):
    bank_t = bank.T
    devs = jax.devices()
    ndev = 2 if len(devs) >= 2 and images.shape[0] % (2 * BLOCK_N) == 0 else 1
    mesh = Mesh(np.array(devs[:ndev]), ("d",))

    def _shard(x, w, bt):
        local = _count_rows(x, w, bt)
        return jax.lax.psum(local, "d")

    counts = jax.shard_map(
        _shard, mesh=mesh,
        in_specs=(P("d", None), P(None, None), P(None, None)),
        out_specs=P(None, None), check_vma=False,
    )(images, W, bank_t)
    return (-counts).reshape(K_BANK)


# speedup vs baseline: 2.0407x; 2.0407x over previous
import jax
import jax.numpy as jnp
from jax.experimental import pallas as pl
from jax.experimental.pallas import tpu as pltpu

K_BANK = 2048
BLOCK_N = 8192
CHUNK = 512


def _hist(scores):
    m = jnp.max(scores, axis=-1, keepdims=True)
    return jnp.sum((scores == m).astype(jnp.int32), axis=0, keepdims=True)


def _fused_body(x_ref, w_ref, bt_ref, o_ref, m_ref):
    i = pl.program_id(0)

    @pl.when(i == 0)
    def _merge():
        m_ref[:] = jnp.dot(w_ref[:], bt_ref[:], preferred_element_type=jnp.float32)

    nchunk = BLOCK_N // CHUNK

    def _mm(c):
        x = x_ref[c * CHUNK:(c + 1) * CHUNK, :]
        return jnp.dot(x, m_ref[:], preferred_element_type=jnp.float32)

    LOOKAHEAD = 2
    pending = [_mm(c) for c in range(min(LOOKAHEAD, nchunk))]
    part = jnp.zeros((1, K_BANK), jnp.int32)
    for c in range(nchunk):
        if c + LOOKAHEAD < nchunk:
            pending.append(_mm(c + LOOKAHEAD))
        part += _hist(pending.pop(0))

    @pl.when(i == 0)
    def _init():
        o_ref[:] = part

    @pl.when(i > 0)
    def _acc():
        o_ref[:] += part


def kernel(images, W, bank):
    n = images.shape[0]
    bank_t = bank.T
    grid = (n // BLOCK_N,)
    counts = pl.pallas_call(
        _fused_body,
        grid=grid,
        in_specs=[
            pl.BlockSpec((BLOCK_N, images.shape[1]), lambda i: (i, 0)),
            pl.BlockSpec(W.shape, lambda i: (0, 0)),
            pl.BlockSpec(bank_t.shape, lambda i: (0, 0)),
        ],
        out_specs=pl.BlockSpec((1, K_BANK), lambda i: (0, 0)),
        out_shape=jax.ShapeDtypeStruct((1, K_BANK), jnp.int32),
        scratch_shapes=[pltpu.VMEM((64, K_BANK), jnp.float32)],
    )(images, W, bank_t)
    return (-counts).reshape(K_BANK)
